# Initial kernel scaffold; baseline (speedup 1.0000x reference)
import jax
import jax.numpy as jnp
from jax.experimental import pallas as pl


def kernel(x, w_ac1_c0, w_ac1_c1, w_ac2_c0, w_ac2_c1, w3, w4, w5, wh1, wh2, bh2, wout, bout):
    raise NotImplementedError("write your pallas kernel here")


# speedup vs baseline: 8.3542x; 8.3542x over previous
import functools

import jax
import jax.numpy as jnp
from jax import lax
from jax.experimental import pallas as pl
from jax.experimental.pallas import tpu as pltpu

BN, NPTS, KNN = 8, 1024, 20
BNDIV = float(jnp.sqrt(jnp.float32(1.0 + 1e-5)))
TR = 256
TG = 256
NEG = -3.0e38

DN_NT = (((1,), (1,)), ((), ()))
DN_NN = (((1,), (0,)), ((), ()))


def _bn(x):
    return x / jnp.float32(BNDIV)


def _lrelu(x):
    return jnp.where(x >= 0, x, 0.2 * x)


def _mm(a, b, dn):
    return lax.dot_general(a, b, dimension_numbers=dn,
                           preferred_element_type=jnp.float32)


def _knn_kernel(featct_ref, featt_ref, idx_ref, d_ref):
    fct = featct_ref[0]
    ft = featt_ref[0]
    inner = -2.0 * _mm(ft, fct, DN_NN)
    sq_c = jnp.sum(fct * fct, axis=0, keepdims=True)
    sq_r = jnp.sum(ft * ft, axis=1, keepdims=True)
    d_ref[...] = ((0.0 - sq_c) - inner) - sq_r
    lanes = lax.broadcasted_iota(jnp.int32, (TR, NPTS), 1)
    lanes_k = lax.broadcasted_iota(jnp.int32, (TR, KNN), 1)

    def body(k, acc):
        d = d_ref[...]
        m = jnp.max(d, axis=1, keepdims=True)
        cand = jnp.where(d >= m, lanes, NPTS)
        amin = jnp.min(cand, axis=1, keepdims=True)
        d_ref[...] = jnp.where(lanes == amin, NEG, d)
        return jnp.where(lanes_k == k, amin, acc)

    idx_ref[0] = lax.fori_loop(0, KNN, body,
                               jnp.zeros((TR, KNN), jnp.int32))


def _knn(feat, feat_ct):
    b, n, c = feat.shape
    return pl.pallas_call(
        _knn_kernel,
        grid=(b, n // TR),
        in_specs=[
            pl.BlockSpec((1, c, n), lambda bi, r: (bi, 0, 0)),
            pl.BlockSpec((1, TR, c), lambda bi, r: (bi, r, 0)),
        ],
        out_specs=pl.BlockSpec((1, TR, KNN), lambda bi, r: (bi, r, 0)),
        out_shape=jax.ShapeDtypeStruct((b, n, KNN), jnp.int32),
        scratch_shapes=[pltpu.VMEM((TR, NPTS), jnp.float32)],
    )(feat_ct, feat)


def _gconv_kernel(xf_ref, xt_ref, w_ref, idx_ref, out_ref):
    xf = xf_ref[0]
    ctr = xt_ref[0]
    w = w_ref[...]
    idx = idx_ref[0]
    o_ch = w.shape[0]
    lanes = lax.broadcasted_iota(jnp.int32, (TG, NPTS), 1)
    acc = jnp.full((TG, o_ch), NEG, jnp.float32)
    for k in range(KNN):
        oh = (idx[:, k:k + 1] == lanes).astype(jnp.float32)
        g = _mm(oh, xf, DN_NN)
        cat = jnp.concatenate([g - ctr, ctr], axis=1)
        acc = jnp.maximum(acc, _lrelu(_bn(_mm(cat, w, DN_NT))))
    out_ref[0] = acc


def _gconv(xfeat, idx, w):
    b, n, c = xfeat.shape
    o_ch = w.shape[0]
    return pl.pallas_call(
        _gconv_kernel,
        grid=(b, n // TG),
        in_specs=[
            pl.BlockSpec((1, n, c), lambda bi, t: (bi, 0, 0)),
            pl.BlockSpec((1, TG, c), lambda bi, t: (bi, t, 0)),
            pl.BlockSpec((o_ch, 2 * c), lambda bi, t: (0, 0)),
            pl.BlockSpec((1, TG, KNN), lambda bi, t: (bi, t, 0)),
        ],
        out_specs=pl.BlockSpec((1, TG, o_ch), lambda bi, t: (bi, t, 0)),
        out_shape=jax.ShapeDtypeStruct((b, n, o_ch), jnp.float32),
    )(xfeat, xfeat, w, idx)


def _aconv_kernel(c_in, ucat_ref, ut_ref, w0_ref, w1_ref, idx_ref, out_ref):
    ucat = ucat_ref[0]
    ctf = ut_ref[0][:, :c_in]
    cc = ut_ref[0][:, c_in:c_in + 3]
    w0 = w0_ref[...]
    w1 = w1_ref[...]
    idx = idx_ref[0]
    lanes = lax.broadcasted_iota(jnp.int32, (TG, NPTS), 1)
    acc = jnp.full((TG, 64), NEG, jnp.float32)
    for k in range(KNN):
        oh = (idx[:, k:k + 1] == lanes).astype(jnp.float32)
        g = _mm(oh, ucat, DN_NN)
        gf = g[:, :c_in]
        cg = g[:, c_in:c_in + 3]
        cat = jnp.concatenate([gf - ctf, ctf], axis=1)
        y1 = _lrelu(_bn(_mm(cat, w0, DN_NT)))
        y2 = _mm(y1, w1, DN_NT)
        pn = cg - cc
        o = jnp.zeros((TG, 64), jnp.float32)
        for j in range(3):
            o = o + y2[:, j * 64:(j + 1) * 64] * pn[:, j:j + 1]
        for j in range(3):
            o = o + y2[:, (j + 3) * 64:(j + 4) * 64] * cc[:, j:j + 1]
        acc = jnp.maximum(acc, _lrelu(_bn(o)))
    out_ref[0] = acc


def _aconv(ucat, w0, w1s, idx, c_in):
    b, n, p = ucat.shape
    return pl.pallas_call(
        functools.partial(_aconv_kernel, c_in),
        grid=(b, n // TG),
        in_specs=[
            pl.BlockSpec((1, n, p), lambda bi, t: (bi, 0, 0)),
            pl.BlockSpec((1, TG, p), lambda bi, t: (bi, t, 0)),
            pl.BlockSpec(w0.shape, lambda bi, t: (0, 0)),
            pl.BlockSpec((384, 64), lambda bi, t: (0, 0)),
            pl.BlockSpec((1, TG, KNN), lambda bi, t: (bi, t, 0)),
        ],
        out_specs=pl.BlockSpec((1, TG, 64), lambda bi, t: (bi, t, 0)),
        out_shape=jax.ShapeDtypeStruct((b, n, 64), jnp.float32),
    )(ucat, ucat, w0, w1s, idx)


def _pool_kernel(xc_ref, w5_ref, out_ref):
    x5 = _lrelu(_bn(_mm(xc_ref[0], w5_ref[...], DN_NT)))
    out_ref[0, 0:1, :] = jnp.max(x5, axis=0, keepdims=True)
    out_ref[0, 1:2, :] = jnp.mean(x5, axis=0, keepdims=True)


def _pool(xc, w5):
    b, n, c = xc.shape
    demb = w5.shape[0]
    return pl.pallas_call(
        _pool_kernel,
        grid=(b,),
        in_specs=[
            pl.BlockSpec((1, n, c), lambda bi: (bi, 0, 0)),
            pl.BlockSpec((demb, c), lambda bi: (0, 0)),
        ],
        out_specs=pl.BlockSpec((1, 2, demb), lambda bi: (bi, 0, 0)),
        out_shape=jax.ShapeDtypeStruct((b, 2, demb), jnp.float32),
    )(xc, w5)


def _head_kernel(p_ref, wh1_ref, wh2_ref, bh2_ref, wout_ref, bout_ref,
                 out_ref):
    h = _lrelu(_bn(_mm(p_ref[...], wh1_ref[...], DN_NT)))
    h = _lrelu(_bn(_mm(h, wh2_ref[...], DN_NT) + bh2_ref[...]))
    out_ref[...] = _mm(h, wout_ref[...], DN_NT) + bout_ref[...]


def _head(pooled, wh1, wh2, bh2, wout, bout):
    b = pooled.shape[0]
    ncls = wout.shape[0]
    return pl.pallas_call(
        _head_kernel,
        out_shape=jax.ShapeDtypeStruct((b, ncls), jnp.float32),
    )(pooled, wh1, wh2, bh2.reshape(1, -1), wout, bout.reshape(1, -1))


def _jmajor(w1):
    return w1.reshape(64, 6, 64).transpose(1, 0, 2).reshape(384, 64)


def kernel(x, w_ac1_c0, w_ac1_c1, w_ac2_c0, w_ac2_c1, w3, w4, w5,
           wh1, wh2, bh2, wout, bout):
    b, _, n = x.shape
    xt = jnp.transpose(x, (0, 2, 1))

    idx1 = _knn(xt, x)
    ucat1 = jnp.concatenate([xt, xt, jnp.zeros((b, n, 2), jnp.float32)],
                            axis=-1)
    x1 = _aconv(ucat1, w_ac1_c0, _jmajor(w_ac1_c1), idx1, 3)

    idx2 = _knn(x1, jnp.transpose(x1, (0, 2, 1)))
    ucat2 = jnp.concatenate([x1, xt, jnp.zeros((b, n, 61), jnp.float32)],
                            axis=-1)
    x2 = _aconv(ucat2, w_ac2_c0, _jmajor(w_ac2_c1), idx2, 64)

    idx3 = _knn(x2, jnp.transpose(x2, (0, 2, 1)))
    x3 = _gconv(x2, idx3, w3)

    idx4 = _knn(x3, jnp.transpose(x3, (0, 2, 1)))
    x4 = _gconv(x3, idx4, w4)

    xc = jnp.concatenate([x1, x2, x3, x4], axis=-1)
    pooled = _pool(xc, w5).reshape(b, 2 * w5.shape[0])
    return _head(pooled, wh1, wh2, bh2, wout, bout)
